# baseline (speedup 1.0000x reference)
import jax
import jax.numpy as jnp
from jax import lax
from jax.experimental import pallas as pl
from jax.experimental.pallas import tpu as pltpu
from jax.experimental.pallas import tpu_sc as plsc

N = 10000
E = 320000
D = 128
DA = 144
NW = 32
CHUNK = 56
NBUF = 4
PH = 36
NPH0 = 4
NPH1 = 6
NCHT = (NPH0 + NPH1) * PH // 2
EP = NW * NCHT * CHUNK
DUMP = 10240
NR = 10368
IR = NR // 16
NC = 10240
CR = NC // 16
BLK = 1000


def _prep_body(x_ref, ps_ref, pc_ref, wxa_ref, wpa_ref, wp_ref, b_ref,
               u_ref, t_ref):
    u = jnp.dot(x_ref[0], wxa_ref[...], preferred_element_type=jnp.float32)
    wpa = wpa_ref[...]
    wp = wp_ref[...]
    ps = ps_ref[0]
    pc = pc_ref[0]
    tcol = b_ref[...]
    for k in range(3):
        u = u + ps[:, k : k + 1] * wpa[k : k + 1, :]
        tcol = tcol - pc[:, k : k + 1] * wp[k : k + 1, :]
    lane = lax.broadcasted_iota(jnp.int32, (BLK, DA), 1)
    u_ref[...] = u + (lane == D).astype(jnp.float32)
    t_ref[...] = tcol


def _prep(x, pos, wxa, wpa, wp, b2):
    return pl.pallas_call(
        _prep_body,
        grid=(N // BLK,),
        in_specs=[
            pl.BlockSpec((1, BLK, D), lambda i: (0, i, 0)),
            pl.BlockSpec((1, BLK, 3), lambda i: (0, i, 0)),
            pl.BlockSpec((1, BLK, 3), lambda i: (1, i, 0)),
            pl.BlockSpec((D, DA), lambda i: (0, 0)),
            pl.BlockSpec((8, DA), lambda i: (0, 0)),
            pl.BlockSpec((8, D), lambda i: (0, 0)),
            pl.BlockSpec((1, D), lambda i: (0, 0)),
        ],
        out_specs=[
            pl.BlockSpec((BLK, DA), lambda i: (i, 0)),
            pl.BlockSpec((BLK, D), lambda i: (i, 0)),
        ],
        out_shape=[
            jax.ShapeDtypeStruct((N, DA), jnp.float32),
            jax.ShapeDtypeStruct((N, D), jnp.float32),
        ],
    )(x, pos, pos, wxa, wpa, wp, b2)


def _sc_body(u_hbm, e_hbm, p0_hbm, p1_hbm,
             src_v, dst_v, buf0, buf1, buf2, buf3, acc, gsem, ssem):
    c = lax.axis_index("c")
    s = lax.axis_index("s")
    wid = c * 16 + s

    zero16 = jnp.zeros((16,), jnp.float32)

    def fill(r, carry):
        for k in range(DA // 16):
            buf0[r, pl.ds(k * 16, 16)] = zero16
        return carry

    lax.fori_loop(0, CHUNK, fill, 0)

    zb = s * IR
    for j in range(IR // CHUNK):
        pltpu.async_copy(buf0, acc.at[pl.ds(zb + j * CHUNK, CHUNK)], gsem)
    pltpu.async_copy(buf0.at[pl.ds(0, IR % CHUNK)],
                     acc.at[pl.ds(zb + IR - IR % CHUNK, IR % CHUNK)], gsem)
    for j in range(IR // CHUNK):
        pltpu.make_async_copy(buf0, acc.at[pl.ds(zb, CHUNK)], gsem).wait()
    pltpu.make_async_copy(buf0.at[pl.ds(0, IR % CHUNK)],
                          acc.at[pl.ds(zb, IR % CHUNK)], gsem).wait()
    plsc.subcore_barrier()

    bufs = (buf0, buf1, buf2, buf3)

    def gather(j, b):
        pltpu.async_copy(u_hbm.at[src_v.at[j]], b, gsem)

    def gwait():
        pltpu.make_async_copy(u_hbm.at[src_v.at[0]], buf0, gsem).wait()

    def scat(j, b):
        pltpu.async_copy(b, acc.at[dst_v.at[j]], ssem, add=True)

    def swait():
        pltpu.make_async_copy(buf0, acc.at[dst_v.at[0]], ssem).wait()

    def group(g, carry):
        for p in range(NBUF):
            j = NBUF * g + p
            gwait()
            if p < 2:
                @pl.when(g > 0)
                def _():
                    swait()
                    gather(j + 2, bufs[p + 2])
            else:
                swait()

                @pl.when(g < PH // NBUF - 1)
                def _():
                    gather(j + 2, bufs[p - 2])
            scat(j, bufs[p])
        return carry

    pbase = jnp.where(c == 0, s * NPH0, 16 * NPH0 + s * NPH1)
    nph = jnp.where(c == 0, NPH0, NPH1)

    def phase(ph, carry):
        base = (pbase + ph) * PH
        pltpu.sync_copy(e_hbm.at[0, pl.ds(base, PH)], src_v)
        pltpu.sync_copy(e_hbm.at[1, pl.ds(base, PH)], dst_v)
        for b in range(NBUF):
            gather(b, bufs[b])
        lax.fori_loop(0, PH // NBUF, group, 0)
        swait()
        swait()
        return carry

    lax.fori_loop(0, nph, phase, 0)

    plsc.subcore_barrier()

    nb = s * CR

    @pl.when(c == 0)
    def _():
        pltpu.sync_copy(acc.at[pl.ds(nb, CR)], p0_hbm.at[pl.ds(nb, CR)])

    @pl.when(c == 1)
    def _():
        pltpu.sync_copy(acc.at[pl.ds(nb, CR)], p1_hbm.at[pl.ds(nb, CR)])


def _scatter(u, e_pad):
    mesh = plsc.VectorSubcoreMesh(
        core_axis_name="c", subcore_axis_name="s", num_cores=2, num_subcores=16
    )
    f = pl.kernel(
        _sc_body,
        mesh=mesh,
        out_type=[
            jax.ShapeDtypeStruct((NC, DA), jnp.float32),
            jax.ShapeDtypeStruct((NC, DA), jnp.float32),
        ],
        scratch_types=[
            pltpu.VMEM((PH, CHUNK), jnp.int32),
            pltpu.VMEM((PH, CHUNK), jnp.int32),
            pltpu.VMEM((CHUNK, DA), jnp.float32),
            pltpu.VMEM((CHUNK, DA), jnp.float32),
            pltpu.VMEM((CHUNK, DA), jnp.float32),
            pltpu.VMEM((CHUNK, DA), jnp.float32),
            pltpu.VMEM_SHARED((NR, DA), jnp.float32),
            pltpu.SemaphoreType.DMA,
            pltpu.SemaphoreType.DMA,
        ],
        compiler_params=pltpu.CompilerParams(use_tc_tiling_on_sc=False),
    )
    return f(u, e_pad)


def _combine_body(p0_ref, p1_ref, t_ref, o_ref):
    p0 = p0_ref[...]
    p1 = p1_ref[...]
    cnt = p0[:, D : D + 1] + p1[:, D : D + 1]
    o_ref[...] = p0[:, :D] + p1[:, :D] + cnt * t_ref[...]


def _combine(p0, p1, t):
    return pl.pallas_call(
        _combine_body,
        grid=(N // BLK,),
        in_specs=[
            pl.BlockSpec((BLK, DA), lambda i: (i, 0)),
            pl.BlockSpec((BLK, DA), lambda i: (i, 0)),
            pl.BlockSpec((BLK, D), lambda i: (i, 0)),
        ],
        out_specs=pl.BlockSpec((BLK, D), lambda i: (i, 0)),
        out_shape=jax.ShapeDtypeStruct((N, D), jnp.float32),
    )(p0, p1, t)


@jax.jit
def kernel(x, pos, edge_index, W, b):
    wxa = jnp.pad(W[:D], ((0, 0), (0, DA - D)))
    wpa = jnp.pad(W[D:], ((0, 5), (0, DA - D)))
    wp = jnp.pad(W[D:], ((0, 5), (0, 0)))
    b2 = b[None, :]

    u, t = _prep(x, pos, wxa, wpa, wp, b2)

    pad = jnp.concatenate(
        [jnp.zeros((1, EP - E), jnp.int32),
         jnp.full((1, EP - E), DUMP, jnp.int32)], axis=0)
    e_pad = jnp.concatenate([edge_index, pad], axis=1).reshape(2, EP // CHUNK, CHUNK)

    p0, p1 = _scatter(u, e_pad)
    return _combine(p0, p1, t)


# speedup vs baseline: 1.1168x; 1.1168x over previous
import jax
import jax.numpy as jnp
from jax import lax
from jax.experimental import pallas as pl
from jax.experimental.pallas import tpu as pltpu
from jax.experimental.pallas import tpu_sc as plsc

N = 10000
E = 320000
D = 128
DA = 144
NW = 32
CHUNK = 56
NBUF = 4
PH = 36
NPH0 = 6
NPH1 = 4
NCHT = (NPH0 + NPH1) * PH // 2
EP = NW * NCHT * CHUNK
DUMP = 10240
NR = 10368
IR = NR // 16
NC = 10240
CR = NC // 16
BLK = 1000


def _prep_body(x_ref, ps_ref, pc_ref, wxa_ref, wpa_ref, wp_ref, b_ref,
               u_ref, t_ref):
    u = jnp.dot(x_ref[0], wxa_ref[...], preferred_element_type=jnp.float32)
    wpa = wpa_ref[...]
    wp = wp_ref[...]
    ps = ps_ref[0]
    pc = pc_ref[0]
    tcol = b_ref[...]
    for k in range(3):
        u = u + ps[:, k : k + 1] * wpa[k : k + 1, :]
        tcol = tcol - pc[:, k : k + 1] * wp[k : k + 1, :]
    lane = lax.broadcasted_iota(jnp.int32, (BLK, DA), 1)
    u_ref[...] = u + (lane == D).astype(jnp.float32)
    t_ref[...] = tcol


def _prep(x, pos, wxa, wpa, wp, b2):
    return pl.pallas_call(
        _prep_body,
        grid=(N // BLK,),
        in_specs=[
            pl.BlockSpec((1, BLK, D), lambda i: (0, i, 0)),
            pl.BlockSpec((1, BLK, 3), lambda i: (0, i, 0)),
            pl.BlockSpec((1, BLK, 3), lambda i: (1, i, 0)),
            pl.BlockSpec((D, DA), lambda i: (0, 0)),
            pl.BlockSpec((8, DA), lambda i: (0, 0)),
            pl.BlockSpec((8, D), lambda i: (0, 0)),
            pl.BlockSpec((1, D), lambda i: (0, 0)),
        ],
        out_specs=[
            pl.BlockSpec((BLK, DA), lambda i: (i, 0)),
            pl.BlockSpec((BLK, D), lambda i: (i, 0)),
        ],
        out_shape=[
            jax.ShapeDtypeStruct((N, DA), jnp.float32),
            jax.ShapeDtypeStruct((N, D), jnp.float32),
        ],
    )(x, pos, pos, wxa, wpa, wp, b2)


def _sc_body(u_hbm, e_hbm, p0_hbm, p1_hbm,
             src_v, dst_v, buf0, buf1, buf2, buf3, acc, gsem, ssem):
    c = lax.axis_index("c")
    s = lax.axis_index("s")
    wid = c * 16 + s

    zero16 = jnp.zeros((16,), jnp.float32)

    def fill(r, carry):
        for k in range(DA // 16):
            buf0[r, pl.ds(k * 16, 16)] = zero16
        return carry

    lax.fori_loop(0, CHUNK, fill, 0)

    zb = s * IR
    for j in range(IR // CHUNK):
        pltpu.async_copy(buf0, acc.at[pl.ds(zb + j * CHUNK, CHUNK)], gsem)
    pltpu.async_copy(buf0.at[pl.ds(0, IR % CHUNK)],
                     acc.at[pl.ds(zb + IR - IR % CHUNK, IR % CHUNK)], gsem)
    for j in range(IR // CHUNK):
        pltpu.make_async_copy(buf0, acc.at[pl.ds(zb, CHUNK)], gsem).wait()
    pltpu.make_async_copy(buf0.at[pl.ds(0, IR % CHUNK)],
                          acc.at[pl.ds(zb, IR % CHUNK)], gsem).wait()
    plsc.subcore_barrier()

    bufs = (buf0, buf1, buf2, buf3)

    def gather(j, b):
        pltpu.async_copy(u_hbm.at[src_v.at[j]], b, gsem)

    def gwait():
        pltpu.make_async_copy(u_hbm.at[src_v.at[0]], buf0, gsem).wait()

    def scat(j, b):
        pltpu.async_copy(b, acc.at[dst_v.at[j]], ssem, add=True)

    def swait():
        pltpu.make_async_copy(buf0, acc.at[dst_v.at[0]], ssem).wait()

    def group(g, carry):
        for p in range(NBUF):
            j = NBUF * g + p
            gwait()
            if p < 2:
                @pl.when(g > 0)
                def _():
                    swait()
                    gather(j + 2, bufs[p + 2])
            else:
                swait()

                @pl.when(g < PH // NBUF - 1)
                def _():
                    gather(j + 2, bufs[p - 2])
            scat(j, bufs[p])
        return carry

    pbase = jnp.where(c == 0, s * NPH0, 16 * NPH0 + s * NPH1)
    nph = jnp.where(c == 0, NPH0, NPH1)

    def phase(ph, carry):
        base = (pbase + ph) * PH
        pltpu.sync_copy(e_hbm.at[0, pl.ds(base, PH)], src_v)
        pltpu.sync_copy(e_hbm.at[1, pl.ds(base, PH)], dst_v)
        for b in range(NBUF):
            gather(b, bufs[b])
        lax.fori_loop(0, PH // NBUF, group, 0)
        swait()
        swait()
        return carry

    lax.fori_loop(0, nph, phase, 0)

    plsc.subcore_barrier()

    nb = s * CR

    @pl.when(c == 0)
    def _():
        pltpu.sync_copy(acc.at[pl.ds(nb, CR)], p0_hbm.at[pl.ds(nb, CR)])

    @pl.when(c == 1)
    def _():
        pltpu.sync_copy(acc.at[pl.ds(nb, CR)], p1_hbm.at[pl.ds(nb, CR)])


def _scatter(u, e_pad):
    mesh = plsc.VectorSubcoreMesh(
        core_axis_name="c", subcore_axis_name="s", num_cores=2, num_subcores=16
    )
    f = pl.kernel(
        _sc_body,
        mesh=mesh,
        out_type=[
            jax.ShapeDtypeStruct((NC, DA), jnp.float32),
            jax.ShapeDtypeStruct((NC, DA), jnp.float32),
        ],
        scratch_types=[
            pltpu.VMEM((PH, CHUNK), jnp.int32),
            pltpu.VMEM((PH, CHUNK), jnp.int32),
            pltpu.VMEM((CHUNK, DA), jnp.float32),
            pltpu.VMEM((CHUNK, DA), jnp.float32),
            pltpu.VMEM((CHUNK, DA), jnp.float32),
            pltpu.VMEM((CHUNK, DA), jnp.float32),
            pltpu.VMEM_SHARED((NR, DA), jnp.float32),
            pltpu.SemaphoreType.DMA,
            pltpu.SemaphoreType.DMA,
        ],
        compiler_params=pltpu.CompilerParams(use_tc_tiling_on_sc=False),
    )
    return f(u, e_pad)


def _combine_body(p0_ref, p1_ref, t_ref, o_ref):
    p0 = p0_ref[...]
    p1 = p1_ref[...]
    cnt = p0[:, D : D + 1] + p1[:, D : D + 1]
    o_ref[...] = p0[:, :D] + p1[:, :D] + cnt * t_ref[...]


def _combine(p0, p1, t):
    return pl.pallas_call(
        _combine_body,
        grid=(N // BLK,),
        in_specs=[
            pl.BlockSpec((BLK, DA), lambda i: (i, 0)),
            pl.BlockSpec((BLK, DA), lambda i: (i, 0)),
            pl.BlockSpec((BLK, D), lambda i: (i, 0)),
        ],
        out_specs=pl.BlockSpec((BLK, D), lambda i: (i, 0)),
        out_shape=jax.ShapeDtypeStruct((N, D), jnp.float32),
    )(p0, p1, t)


@jax.jit
def kernel(x, pos, edge_index, W, b):
    wxa = jnp.pad(W[:D], ((0, 0), (0, DA - D)))
    wpa = jnp.pad(W[D:], ((0, 5), (0, DA - D)))
    wp = jnp.pad(W[D:], ((0, 5), (0, 0)))
    b2 = b[None, :]

    u, t = _prep(x, pos, wxa, wpa, wp, b2)

    pad = jnp.concatenate(
        [jnp.zeros((1, EP - E), jnp.int32),
         jnp.full((1, EP - E), DUMP, jnp.int32)], axis=0)
    e_pad = jnp.concatenate([edge_index, pad], axis=1).reshape(2, EP // CHUNK, CHUNK)

    p0, p1 = _scatter(u, e_pad)
    return _combine(p0, p1, t)


# speedup vs baseline: 1.1396x; 1.0204x over previous
import jax
import jax.numpy as jnp
from jax import lax
from jax.experimental import pallas as pl
from jax.experimental.pallas import tpu as pltpu
from jax.experimental.pallas import tpu_sc as plsc

N = 10000
E = 320000
D = 128
DA = 144
NW = 32
CHUNK = 56
NBUF = 4
PH = 36
NPH0 = 7
NPH1 = 3
NCHT = (NPH0 + NPH1) * PH // 2
EP = NW * NCHT * CHUNK
DUMP = 10240
NR = 10368
IR = NR // 16
NC = 10240
CR = NC // 16
BLK = 1000


def _prep_body(x_ref, ps_ref, pc_ref, wxa_ref, wpa_ref, wp_ref, b_ref,
               u_ref, t_ref):
    u = jnp.dot(x_ref[0], wxa_ref[...], preferred_element_type=jnp.float32)
    wpa = wpa_ref[...]
    wp = wp_ref[...]
    ps = ps_ref[0]
    pc = pc_ref[0]
    tcol = b_ref[...]
    for k in range(3):
        u = u + ps[:, k : k + 1] * wpa[k : k + 1, :]
        tcol = tcol - pc[:, k : k + 1] * wp[k : k + 1, :]
    lane = lax.broadcasted_iota(jnp.int32, (BLK, DA), 1)
    u_ref[...] = u + (lane == D).astype(jnp.float32)
    t_ref[...] = tcol


def _prep(x, pos, wxa, wpa, wp, b2):
    return pl.pallas_call(
        _prep_body,
        grid=(N // BLK,),
        in_specs=[
            pl.BlockSpec((1, BLK, D), lambda i: (0, i, 0)),
            pl.BlockSpec((1, BLK, 3), lambda i: (0, i, 0)),
            pl.BlockSpec((1, BLK, 3), lambda i: (1, i, 0)),
            pl.BlockSpec((D, DA), lambda i: (0, 0)),
            pl.BlockSpec((8, DA), lambda i: (0, 0)),
            pl.BlockSpec((8, D), lambda i: (0, 0)),
            pl.BlockSpec((1, D), lambda i: (0, 0)),
        ],
        out_specs=[
            pl.BlockSpec((BLK, DA), lambda i: (i, 0)),
            pl.BlockSpec((BLK, D), lambda i: (i, 0)),
        ],
        out_shape=[
            jax.ShapeDtypeStruct((N, DA), jnp.float32),
            jax.ShapeDtypeStruct((N, D), jnp.float32),
        ],
    )(x, pos, pos, wxa, wpa, wp, b2)


def _sc_body(u_hbm, e_hbm, p0_hbm, p1_hbm,
             src_v, dst_v, buf0, buf1, buf2, buf3, acc, gsem, ssem):
    c = lax.axis_index("c")
    s = lax.axis_index("s")
    wid = c * 16 + s

    zero16 = jnp.zeros((16,), jnp.float32)

    def fill(r, carry):
        for k in range(DA // 16):
            buf0[r, pl.ds(k * 16, 16)] = zero16
        return carry

    lax.fori_loop(0, CHUNK, fill, 0)

    zb = s * IR
    for j in range(IR // CHUNK):
        pltpu.async_copy(buf0, acc.at[pl.ds(zb + j * CHUNK, CHUNK)], gsem)
    pltpu.async_copy(buf0.at[pl.ds(0, IR % CHUNK)],
                     acc.at[pl.ds(zb + IR - IR % CHUNK, IR % CHUNK)], gsem)
    for j in range(IR // CHUNK):
        pltpu.make_async_copy(buf0, acc.at[pl.ds(zb, CHUNK)], gsem).wait()
    pltpu.make_async_copy(buf0.at[pl.ds(0, IR % CHUNK)],
                          acc.at[pl.ds(zb, IR % CHUNK)], gsem).wait()
    plsc.subcore_barrier()

    bufs = (buf0, buf1, buf2, buf3)

    def gather(j, b):
        pltpu.async_copy(u_hbm.at[src_v.at[j]], b, gsem)

    def gwait():
        pltpu.make_async_copy(u_hbm.at[src_v.at[0]], buf0, gsem).wait()

    def scat(j, b):
        pltpu.async_copy(b, acc.at[dst_v.at[j]], ssem, add=True)

    def swait():
        pltpu.make_async_copy(buf0, acc.at[dst_v.at[0]], ssem).wait()

    def group(g, carry):
        for p in range(NBUF):
            j = NBUF * g + p
            gwait()
            if p < 2:
                @pl.when(g > 0)
                def _():
                    swait()
                    gather(j + 2, bufs[p + 2])
            else:
                swait()

                @pl.when(g < PH // NBUF - 1)
                def _():
                    gather(j + 2, bufs[p - 2])
            scat(j, bufs[p])
        return carry

    pbase = jnp.where(c == 0, s * NPH0, 16 * NPH0 + s * NPH1)
    nph = jnp.where(c == 0, NPH0, NPH1)

    def phase(ph, carry):
        base = (pbase + ph) * PH
        pltpu.sync_copy(e_hbm.at[0, pl.ds(base, PH)], src_v)
        pltpu.sync_copy(e_hbm.at[1, pl.ds(base, PH)], dst_v)
        for b in range(NBUF):
            gather(b, bufs[b])
        lax.fori_loop(0, PH // NBUF, group, 0)
        swait()
        swait()
        return carry

    lax.fori_loop(0, nph, phase, 0)

    plsc.subcore_barrier()

    nb = s * CR

    @pl.when(c == 0)
    def _():
        pltpu.sync_copy(acc.at[pl.ds(nb, CR)], p0_hbm.at[pl.ds(nb, CR)])

    @pl.when(c == 1)
    def _():
        pltpu.sync_copy(acc.at[pl.ds(nb, CR)], p1_hbm.at[pl.ds(nb, CR)])


def _scatter(u, e_pad):
    mesh = plsc.VectorSubcoreMesh(
        core_axis_name="c", subcore_axis_name="s", num_cores=2, num_subcores=16
    )
    f = pl.kernel(
        _sc_body,
        mesh=mesh,
        out_type=[
            jax.ShapeDtypeStruct((NC, DA), jnp.float32),
            jax.ShapeDtypeStruct((NC, DA), jnp.float32),
        ],
        scratch_types=[
            pltpu.VMEM((PH, CHUNK), jnp.int32),
            pltpu.VMEM((PH, CHUNK), jnp.int32),
            pltpu.VMEM((CHUNK, DA), jnp.float32),
            pltpu.VMEM((CHUNK, DA), jnp.float32),
            pltpu.VMEM((CHUNK, DA), jnp.float32),
            pltpu.VMEM((CHUNK, DA), jnp.float32),
            pltpu.VMEM_SHARED((NR, DA), jnp.float32),
            pltpu.SemaphoreType.DMA,
            pltpu.SemaphoreType.DMA,
        ],
        compiler_params=pltpu.CompilerParams(use_tc_tiling_on_sc=False),
    )
    return f(u, e_pad)


def _combine_body(p0_ref, p1_ref, t_ref, o_ref):
    p0 = p0_ref[...]
    p1 = p1_ref[...]
    cnt = p0[:, D : D + 1] + p1[:, D : D + 1]
    o_ref[...] = p0[:, :D] + p1[:, :D] + cnt * t_ref[...]


def _combine(p0, p1, t):
    return pl.pallas_call(
        _combine_body,
        grid=(N // BLK,),
        in_specs=[
            pl.BlockSpec((BLK, DA), lambda i: (i, 0)),
            pl.BlockSpec((BLK, DA), lambda i: (i, 0)),
            pl.BlockSpec((BLK, D), lambda i: (i, 0)),
        ],
        out_specs=pl.BlockSpec((BLK, D), lambda i: (i, 0)),
        out_shape=jax.ShapeDtypeStruct((N, D), jnp.float32),
    )(p0, p1, t)


@jax.jit
def kernel(x, pos, edge_index, W, b):
    wxa = jnp.pad(W[:D], ((0, 0), (0, DA - D)))
    wpa = jnp.pad(W[D:], ((0, 5), (0, DA - D)))
    wp = jnp.pad(W[D:], ((0, 5), (0, 0)))
    b2 = b[None, :]

    u, t = _prep(x, pos, wxa, wpa, wp, b2)

    pad = jnp.concatenate(
        [jnp.zeros((1, EP - E), jnp.int32),
         jnp.full((1, EP - E), DUMP, jnp.int32)], axis=0)
    e_pad = jnp.concatenate([edge_index, pad], axis=1).reshape(2, EP // CHUNK, CHUNK)

    p0, p1 = _scatter(u, e_pad)
    return _combine(p0, p1, t)
